# baseline (speedup 1.0000x reference)
import functools

import jax
import jax.numpy as jnp
from jax import lax
from jax.experimental import pallas as pl
from jax.experimental.pallas import tpu as pltpu
from jax.experimental.pallas import tpu_sc as plsc

_N = 10000
_E = 320000
_D = 128
_H = 128
_G = 16

_NC = 2
_NS = 16
_NW = _NC * _NS
_EPT = _E // _NW
_K = 80
_NCHUNK = 128
_EPTP = _K * _NCHUNK
_NP = 10112
_RPT = _NP // _NS

_R = 2000
_NB = _N // _R


def _sc_segsum(x, src1, dst1, dst3, zeros, variant):
    mesh = plsc.VectorSubcoreMesh(core_axis_name="c", subcore_axis_name="s")

    if variant == 1:
        scratch = [
            pltpu.VMEM((_K,), jnp.int32),
            pltpu.VMEM((_K,), jnp.int32),
            pltpu.VMEM((_K, _D), jnp.float32),
            pltpu.VMEM_SHARED((_NP, _D), jnp.float32),
            pltpu.SemaphoreType.DMA,
        ]
    elif variant == 2:
        scratch = [
            pltpu.VMEM((_EPTP,), jnp.int32),
            pltpu.VMEM((_NCHUNK, _K), jnp.int32),
            pltpu.VMEM((_K, _D), jnp.float32),
            pltpu.VMEM_SHARED((_NP, _D), jnp.float32),
            pltpu.SemaphoreType.DMA,
        ]
    else:
        scratch = [
            pltpu.VMEM((_EPTP,), jnp.int32),
            pltpu.VMEM((_NCHUNK, _K), jnp.int32),
            pltpu.VMEM((_K, _D), jnp.float32),
            pltpu.VMEM((_K, _D), jnp.float32),
            pltpu.VMEM_SHARED((_NP, _D), jnp.float32),
            pltpu.SemaphoreType.DMA,
            pltpu.SemaphoreType.DMA,
        ]

    @functools.partial(
        pl.kernel,
        mesh=mesh,
        out_type=jax.ShapeDtypeStruct((_NC, _NP, _D), jnp.float32),
        scratch_types=scratch,
    )
    def k(x_hbm, src_hbm, dst_hbm, dst3_hbm, z_hbm, out_hbm, *sc):
        cid = lax.axis_index("c")
        sid = lax.axis_index("s")
        wid = sid * _NC + cid
        r0 = sid * _RPT
        agg = sc[-2] if variant != 3 else sc[-3]
        pltpu.sync_copy(z_hbm, agg.at[pl.ds(r0, _RPT)])

        if variant == 1:
            isrc, idst, rows, _, sem = sc
            plsc.subcore_barrier()

            def body(j, carry):
                e0 = wid * _EPTP + j * _K
                pltpu.sync_copy(src_hbm.at[pl.ds(e0, _K)], isrc)
                pltpu.sync_copy(dst_hbm.at[pl.ds(e0, _K)], idst)
                pltpu.async_copy(x_hbm.at[isrc], rows, sem).wait()
                pltpu.sync_copy(rows, agg.at[idst], add=True)
                return carry

            lax.fori_loop(0, _NCHUNK, body, 0)
        elif variant == 2:
            isrc, idst, rows, _, sem = sc
            pltpu.sync_copy(src_hbm.at[pl.ds(wid * _EPTP, _EPTP)], isrc)
            pltpu.sync_copy(dst3_hbm.at[wid], idst)
            plsc.subcore_barrier()

            def body(j, carry):
                pltpu.async_copy(x_hbm.at[isrc.at[pl.ds(j * _K, _K)]], rows,
                                 sem).wait()
                pltpu.sync_copy(rows, agg.at[idst.at[j]], add=True)
                return carry

            lax.fori_loop(0, _NCHUNK, body, 0)
        else:
            isrc, idst, rows_a, rows_b, _, sem_a, sem_b = sc
            pltpu.sync_copy(src_hbm.at[pl.ds(wid * _EPTP, _EPTP)], isrc)
            pltpu.sync_copy(dst3_hbm.at[wid], idst)
            plsc.subcore_barrier()

            def sidx(j):
                return isrc.at[pl.ds(j * _K, _K)]

            pltpu.async_copy(x_hbm.at[sidx(0)], rows_a, sem_a)

            def body(i, carry):
                j = 2 * i
                pltpu.make_async_copy(x_hbm.at[sidx(j)], rows_a,
                                      sem_a).wait()
                pltpu.async_copy(x_hbm.at[sidx(j + 1)], rows_b, sem_b)
                pltpu.sync_copy(rows_a, agg.at[idst.at[j]], add=True)
                pltpu.make_async_copy(x_hbm.at[sidx(j + 1)], rows_b,
                                      sem_b).wait()

                @pl.when(j + 2 < _NCHUNK)
                def _():
                    pltpu.async_copy(x_hbm.at[sidx(j + 2)], rows_a, sem_a)

                pltpu.sync_copy(rows_b, agg.at[idst.at[j + 1]], add=True)
                return carry

            lax.fori_loop(0, _NCHUNK // 2, body, 0)

        plsc.subcore_barrier()
        pltpu.sync_copy(agg.at[pl.ds(r0, _RPT)],
                        out_hbm.at[cid].at[pl.ds(r0, _RPT)])

    return k(x, src1, dst1, dst3, zeros)


def _mlp_body(x_ref, p0_ref, p1_ref, w1_ref, b1_ref, w2_ref, b2_ref, eps_ref,
              z_ref, s_ref, ss_ref):
    i = pl.program_id(0)
    h = (1.0 + eps_ref[0, 0]) * x_ref[...] + p0_ref[0] + p1_ref[0]
    a = jnp.maximum(
        jnp.dot(h, w1_ref[...], preferred_element_type=jnp.float32)
        + b1_ref[...], 0.0)
    z = jnp.maximum(
        jnp.dot(a, w2_ref[...], preferred_element_type=jnp.float32)
        + b2_ref[...], 0.0)
    z_ref[...] = z

    @pl.when(i == 0)
    def _():
        s_ref[...] = jnp.zeros_like(s_ref)
        ss_ref[...] = jnp.zeros_like(ss_ref)

    s_ref[...] += jnp.sum(z, axis=0, keepdims=True)
    ss_ref[...] += jnp.sum(z * z, axis=0, keepdims=True)


def _mlp(x, p, w1, b1, w2, b2, eps):
    return pl.pallas_call(
        _mlp_body,
        grid=(_NB,),
        in_specs=[
            pl.BlockSpec((_R, _D), lambda i: (i, 0)),
            pl.BlockSpec((1, _R, _D), lambda i: (0, i, 0)),
            pl.BlockSpec((1, _R, _D), lambda i: (1, i, 0)),
            pl.BlockSpec((_D, _H), lambda i: (0, 0)),
            pl.BlockSpec((1, _H), lambda i: (0, 0)),
            pl.BlockSpec((_H, _H), lambda i: (0, 0)),
            pl.BlockSpec((1, _H), lambda i: (0, 0)),
            pl.BlockSpec((1, 1), lambda i: (0, 0)),
        ],
        out_specs=[
            pl.BlockSpec((_R, _H), lambda i: (i, 0)),
            pl.BlockSpec((1, _H), lambda i: (0, 0)),
            pl.BlockSpec((1, _H), lambda i: (0, 0)),
        ],
        out_shape=[
            jax.ShapeDtypeStruct((_N, _H), jnp.float32),
            jax.ShapeDtypeStruct((1, _H), jnp.float32),
            jax.ShapeDtypeStruct((1, _H), jnp.float32),
        ],
    )(x, p, p, w1, b1, w2, b2, eps)


def _norm_body(z_ref, s_ref, ss_ref, g_ref, be_ref, xn_ref):
    m = s_ref[...] / _N
    v = ss_ref[...] / _N - m * m
    inv = 1.0 / jnp.sqrt(v + 1e-5)
    xn_ref[...] = (z_ref[...] - m) * (inv * g_ref[...]) + be_ref[...]


def _norm(z, s, ss, g, be):
    return pl.pallas_call(
        _norm_body,
        grid=(_NB,),
        in_specs=[
            pl.BlockSpec((_R, _H), lambda i: (i, 0)),
            pl.BlockSpec((1, _H), lambda i: (0, 0)),
            pl.BlockSpec((1, _H), lambda i: (0, 0)),
            pl.BlockSpec((1, _H), lambda i: (0, 0)),
            pl.BlockSpec((1, _H), lambda i: (0, 0)),
        ],
        out_specs=pl.BlockSpec((_R, _H), lambda i: (i, 0)),
        out_shape=jax.ShapeDtypeStruct((_N, _H), jnp.float32),
    )(z, s, ss, g, be)


def _norm_pool_body(z_ref, s_ref, ss_ref, g_ref, be_ref, b_ref, out_ref,
                    acc_ref, cnt_ref):
    i = pl.program_id(0)
    m = s_ref[...] / _N
    v = ss_ref[...] / _N - m * m
    inv = 1.0 / jnp.sqrt(v + 1e-5)
    xn = (z_ref[...] - m) * (inv * g_ref[...]) + be_ref[...]
    bf = b_ref[0, 0, :]
    onehot = (bf[:, None]
              == lax.broadcasted_iota(jnp.int32, (_R, _G), 1).astype(jnp.float32)
              ).astype(jnp.float32)

    @pl.when(i == 0)
    def _():
        acc_ref[...] = jnp.zeros_like(acc_ref)
        cnt_ref[...] = jnp.zeros_like(cnt_ref)

    acc_ref[...] += lax.dot_general(
        onehot, xn, (((0,), (0,)), ((), ())),
        preferred_element_type=jnp.float32, precision=lax.Precision.HIGHEST)
    cnt_ref[...] += lax.dot_general(
        onehot, jnp.ones((_R, _H), jnp.float32), (((0,), (0,)), ((), ())),
        preferred_element_type=jnp.float32, precision=lax.Precision.HIGHEST)

    @pl.when(i == _NB - 1)
    def _():
        out_ref[...] = acc_ref[...] / jnp.maximum(cnt_ref[...], 1.0)


def _norm_pool(z, s, ss, g, be, batch3):
    return pl.pallas_call(
        _norm_pool_body,
        grid=(_NB,),
        in_specs=[
            pl.BlockSpec((_R, _H), lambda i: (i, 0)),
            pl.BlockSpec((1, _H), lambda i: (0, 0)),
            pl.BlockSpec((1, _H), lambda i: (0, 0)),
            pl.BlockSpec((1, _H), lambda i: (0, 0)),
            pl.BlockSpec((1, _H), lambda i: (0, 0)),
            pl.BlockSpec((1, 1, _R), lambda i: (i, 0, 0)),
        ],
        out_specs=pl.BlockSpec((_G, _H), lambda i: (0, 0)),
        out_shape=jax.ShapeDtypeStruct((_G, _H), jnp.float32),
        scratch_shapes=[
            pltpu.VMEM((_G, _H), jnp.float32),
            pltpu.VMEM((_G, _H), jnp.float32),
        ],
    )(z, s, ss, g, be, batch3)


def kernel(x, edge_index, batch,
           w1_0, b1_0, w2_0, b2_0, g_0, be_0, eps_0,
           w1_1, b1_1, w2_1, b2_1, g_1, be_1, eps_1,
           w1_2, b1_2, w2_2, b2_2, g_2, be_2, eps_2):
    pad = _EPTP - _EPT
    src1 = jnp.pad(edge_index[0].reshape(_NW, _EPT), ((0, 0), (0, pad)),
                   constant_values=0).reshape(_NW * _EPTP)
    dstp = jnp.pad(edge_index[1].reshape(_NW, _EPT), ((0, 0), (0, pad)),
                   constant_values=_NP - 1)
    dst1 = dstp.reshape(_NW * _EPTP)
    dst3 = dstp.reshape(_NW, _NCHUNK, _K)
    zeros = jnp.zeros((_RPT, _D), jnp.float32)
    batch3 = batch.astype(jnp.float32).reshape(_NB, 1, _R)

    layers = [
        (w1_0, b1_0, w2_0, b2_0, g_0, be_0, eps_0),
        (w1_1, b1_1, w2_1, b2_1, g_1, be_1, eps_1),
        (w1_2, b1_2, w2_2, b2_2, g_2, be_2, eps_2),
    ]
    h = x
    for li, (w1, b1, w2, b2, g, be, eps) in enumerate(layers):
        p = _sc_segsum(h, src1, dst1, dst3, zeros, variant=li + 1)
        z, s, ss = _mlp(h, p, w1, b1.reshape(1, _H), w2, b2.reshape(1, _H),
                        eps.reshape(1, 1))
        if li < 2:
            h = _norm(z, s, ss, g.reshape(1, _H), be.reshape(1, _H))
        else:
            out = _norm_pool(z, s, ss, g.reshape(1, _H), be.reshape(1, _H),
                             batch3)
    return out


# speedup vs baseline: 2.8777x; 2.8777x over previous
import functools

import jax
import jax.numpy as jnp
from jax import lax
from jax.experimental import pallas as pl
from jax.experimental.pallas import tpu as pltpu
from jax.experimental.pallas import tpu_sc as plsc

_N = 10000
_E = 320000
_D = 128
_H = 128
_G = 16

_NC = 2
_NS = 16
_NW = _NC * _NS
_EPT = _E // _NW
_K = 80
_NCHUNK = _EPT // _K
_NP = 10240
_RPT = _NP // _NS

_R = 2000
_NB = _N // _R


def _sc_segsum(x, src, dst, zeros):
    mesh = plsc.VectorSubcoreMesh(core_axis_name="c", subcore_axis_name="s")

    @functools.partial(
        pl.kernel,
        mesh=mesh,
        out_type=jax.ShapeDtypeStruct((_NC, _NP, _D), jnp.float32),
        scratch_types=[
            pltpu.VMEM((_K,), jnp.int32),
            pltpu.VMEM((_K,), jnp.int32),
            pltpu.VMEM((_K,), jnp.int32),
            pltpu.VMEM((_K,), jnp.int32),
            pltpu.VMEM((_K, _D), jnp.float32),
            pltpu.VMEM((_K, _D), jnp.float32),
            pltpu.VMEM_SHARED((_NP, _D), jnp.float32),
            pltpu.SemaphoreType.DMA,
            pltpu.SemaphoreType.DMA,
            pltpu.SemaphoreType.DMA,
            pltpu.SemaphoreType.DMA,
        ],
    )
    def k(x_hbm, src_hbm, dst_hbm, z_hbm, out_hbm,
          isrc_a, idst_a, isrc_b, idst_b, rows_a, rows_b, agg,
          semi_a, semi_b, semr_a, semr_b):
        cid = lax.axis_index("c")
        sid = lax.axis_index("s")
        wid = sid * _NC + cid
        r0 = sid * _RPT
        e_base = wid * _EPT
        pltpu.sync_copy(z_hbm, agg.at[pl.ds(r0, _RPT)])
        plsc.subcore_barrier()

        def start_idx(j, isrc, idst, sem):
            e0 = e_base + j * _K
            pltpu.async_copy(src_hbm.at[pl.ds(e0, _K)], isrc, sem)
            pltpu.async_copy(dst_hbm.at[pl.ds(e0, _K)], idst, sem)

        def wait_idx(isrc, idst, sem):
            pltpu.make_async_copy(src_hbm.at[pl.ds(0, _K)], isrc, sem).wait()
            pltpu.make_async_copy(dst_hbm.at[pl.ds(0, _K)], idst, sem).wait()

        start_idx(0, isrc_a, idst_a, semi_a)
        start_idx(1, isrc_b, idst_b, semi_b)
        wait_idx(isrc_a, idst_a, semi_a)
        pltpu.async_copy(x_hbm.at[isrc_a], rows_a, semr_a)

        def body(i, carry):
            j = 2 * i
            pltpu.make_async_copy(x_hbm.at[isrc_a], rows_a, semr_a).wait()
            wait_idx(isrc_b, idst_b, semi_b)
            pltpu.async_copy(x_hbm.at[isrc_b], rows_b, semr_b)
            pltpu.sync_copy(rows_a, agg.at[idst_a], add=True)
            start_idx(j + 2, isrc_a, idst_a, semi_a)
            pltpu.make_async_copy(x_hbm.at[isrc_b], rows_b, semr_b).wait()
            wait_idx(isrc_a, idst_a, semi_a)
            pltpu.async_copy(x_hbm.at[isrc_a], rows_a, semr_a)
            pltpu.sync_copy(rows_b, agg.at[idst_b], add=True)

            @pl.when(j + 3 < _NCHUNK)
            def _():
                start_idx(j + 3, isrc_b, idst_b, semi_b)

            return carry

        lax.fori_loop(0, (_NCHUNK - 1) // 2, body, 0)
        pltpu.make_async_copy(x_hbm.at[isrc_a], rows_a, semr_a).wait()
        pltpu.sync_copy(rows_a, agg.at[idst_a], add=True)

        plsc.subcore_barrier()
        pltpu.sync_copy(agg.at[pl.ds(r0, _RPT)],
                        out_hbm.at[cid].at[pl.ds(r0, _RPT)])

    return k(x, src, dst, zeros)


def _mlp_body(x_ref, p0_ref, p1_ref, w1_ref, b1_ref, w2_ref, b2_ref, eps_ref,
              z_ref, s_ref, ss_ref):
    i = pl.program_id(0)
    h = (1.0 + eps_ref[0, 0]) * x_ref[...] + p0_ref[0] + p1_ref[0]
    a = jnp.maximum(
        jnp.dot(h, w1_ref[...], preferred_element_type=jnp.float32)
        + b1_ref[...], 0.0)
    z = jnp.maximum(
        jnp.dot(a, w2_ref[...], preferred_element_type=jnp.float32)
        + b2_ref[...], 0.0)
    z_ref[...] = z

    @pl.when(i == 0)
    def _():
        s_ref[...] = jnp.zeros_like(s_ref)
        ss_ref[...] = jnp.zeros_like(ss_ref)

    s_ref[...] += jnp.sum(z, axis=0, keepdims=True)
    ss_ref[...] += jnp.sum(z * z, axis=0, keepdims=True)


def _mlp(x, p, w1, b1, w2, b2, eps):
    return pl.pallas_call(
        _mlp_body,
        grid=(_NB,),
        in_specs=[
            pl.BlockSpec((_R, _D), lambda i: (i, 0)),
            pl.BlockSpec((1, _R, _D), lambda i: (0, i, 0)),
            pl.BlockSpec((1, _R, _D), lambda i: (1, i, 0)),
            pl.BlockSpec((_D, _H), lambda i: (0, 0)),
            pl.BlockSpec((1, _H), lambda i: (0, 0)),
            pl.BlockSpec((_H, _H), lambda i: (0, 0)),
            pl.BlockSpec((1, _H), lambda i: (0, 0)),
            pl.BlockSpec((1, 1), lambda i: (0, 0)),
        ],
        out_specs=[
            pl.BlockSpec((_R, _H), lambda i: (i, 0)),
            pl.BlockSpec((1, _H), lambda i: (0, 0)),
            pl.BlockSpec((1, _H), lambda i: (0, 0)),
        ],
        out_shape=[
            jax.ShapeDtypeStruct((_N, _H), jnp.float32),
            jax.ShapeDtypeStruct((1, _H), jnp.float32),
            jax.ShapeDtypeStruct((1, _H), jnp.float32),
        ],
    )(x, p, p, w1, b1, w2, b2, eps)


def _norm_body(z_ref, s_ref, ss_ref, g_ref, be_ref, xn_ref):
    m = s_ref[...] / _N
    v = ss_ref[...] / _N - m * m
    inv = 1.0 / jnp.sqrt(v + 1e-5)
    xn_ref[...] = (z_ref[...] - m) * (inv * g_ref[...]) + be_ref[...]


def _norm(z, s, ss, g, be):
    return pl.pallas_call(
        _norm_body,
        grid=(_NB,),
        in_specs=[
            pl.BlockSpec((_R, _H), lambda i: (i, 0)),
            pl.BlockSpec((1, _H), lambda i: (0, 0)),
            pl.BlockSpec((1, _H), lambda i: (0, 0)),
            pl.BlockSpec((1, _H), lambda i: (0, 0)),
            pl.BlockSpec((1, _H), lambda i: (0, 0)),
        ],
        out_specs=pl.BlockSpec((_R, _H), lambda i: (i, 0)),
        out_shape=jax.ShapeDtypeStruct((_N, _H), jnp.float32),
    )(z, s, ss, g, be)


def _norm_pool_body(z_ref, s_ref, ss_ref, g_ref, be_ref, b_ref, out_ref,
                    acc_ref, cnt_ref):
    i = pl.program_id(0)
    m = s_ref[...] / _N
    v = ss_ref[...] / _N - m * m
    inv = 1.0 / jnp.sqrt(v + 1e-5)
    xn = (z_ref[...] - m) * (inv * g_ref[...]) + be_ref[...]
    bf = b_ref[0, 0, :]
    onehot = (bf[:, None]
              == lax.broadcasted_iota(jnp.int32, (_R, _G), 1).astype(jnp.float32)
              ).astype(jnp.float32)

    @pl.when(i == 0)
    def _():
        acc_ref[...] = jnp.zeros_like(acc_ref)
        cnt_ref[...] = jnp.zeros_like(cnt_ref)

    acc_ref[...] += lax.dot_general(
        onehot, xn, (((0,), (0,)), ((), ())),
        preferred_element_type=jnp.float32, precision=lax.Precision.HIGHEST)
    cnt_ref[...] += lax.dot_general(
        onehot, jnp.ones((_R, _H), jnp.float32), (((0,), (0,)), ((), ())),
        preferred_element_type=jnp.float32, precision=lax.Precision.HIGHEST)

    @pl.when(i == _NB - 1)
    def _():
        out_ref[...] = acc_ref[...] / jnp.maximum(cnt_ref[...], 1.0)


def _norm_pool(z, s, ss, g, be, batch3):
    return pl.pallas_call(
        _norm_pool_body,
        grid=(_NB,),
        in_specs=[
            pl.BlockSpec((_R, _H), lambda i: (i, 0)),
            pl.BlockSpec((1, _H), lambda i: (0, 0)),
            pl.BlockSpec((1, _H), lambda i: (0, 0)),
            pl.BlockSpec((1, _H), lambda i: (0, 0)),
            pl.BlockSpec((1, _H), lambda i: (0, 0)),
            pl.BlockSpec((1, 1, _R), lambda i: (i, 0, 0)),
        ],
        out_specs=pl.BlockSpec((_G, _H), lambda i: (0, 0)),
        out_shape=jax.ShapeDtypeStruct((_G, _H), jnp.float32),
        scratch_shapes=[
            pltpu.VMEM((_G, _H), jnp.float32),
            pltpu.VMEM((_G, _H), jnp.float32),
        ],
    )(z, s, ss, g, be, batch3)


def kernel(x, edge_index, batch,
           w1_0, b1_0, w2_0, b2_0, g_0, be_0, eps_0,
           w1_1, b1_1, w2_1, b2_1, g_1, be_1, eps_1,
           w1_2, b1_2, w2_2, b2_2, g_2, be_2, eps_2):
    src = edge_index[0]
    dst = edge_index[1]
    zeros = jnp.zeros((_RPT, _D), jnp.float32)
    batch3 = batch.astype(jnp.float32).reshape(_NB, 1, _R)

    layers = [
        (w1_0, b1_0, w2_0, b2_0, g_0, be_0, eps_0),
        (w1_1, b1_1, w2_1, b2_1, g_1, be_1, eps_1),
        (w1_2, b1_2, w2_2, b2_2, g_2, be_2, eps_2),
    ]
    h = x
    for li, (w1, b1, w2, b2, g, be, eps) in enumerate(layers):
        p = _sc_segsum(h, src, dst, zeros)
        z, s, ss = _mlp(h, p, w1, b1.reshape(1, _H), w2, b2.reshape(1, _H),
                        eps.reshape(1, 1))
        if li < 2:
            h = _norm(z, s, ss, g.reshape(1, _H), be.reshape(1, _H))
        else:
            out = _norm_pool(z, s, ss, g.reshape(1, _H), be.reshape(1, _H),
                             batch3)
    return out
